# Initial kernel scaffold; baseline (speedup 1.0000x reference)
import jax
import jax.numpy as jnp
from jax.experimental import pallas as pl


def kernel(x, edge_index, W1, b1, W2, b2, W3, b3, W4, b4):
    raise NotImplementedError("write your pallas kernel here")


# speedup vs baseline: 17.3878x; 17.3878x over previous
import functools

import jax
import jax.numpy as jnp
from jax import lax
from jax.experimental import pallas as pl
from jax.experimental.pallas import tpu as pltpu
from jax.experimental.pallas import tpu_sc as plsc

NC = 2
NS = 16
NW = NC * NS
L = 16
CHUNK = 128
DEGW = 16

_HI = jax.lax.Precision.HIGHEST


def _make_degree(rows_pad, n_chunks):
    zrows = rows_pad // NS
    mesh = plsc.VectorSubcoreMesh(core_axis_name="c", subcore_axis_name="s")

    @functools.partial(
        pl.kernel,
        out_type=jax.ShapeDtypeStruct((NC, rows_pad), jnp.float32),
        mesh=mesh,
        scratch_types=[
            pltpu.VMEM((n_chunks, CHUNK), jnp.int32),
            pltpu.VMEM((CHUNK,), jnp.float32),
            pltpu.VMEM((CHUNK,), jnp.float32),
            pltpu.VMEM_SHARED((rows_pad,), jnp.float32),
        ],
    )
    def degk(dst_hbm, out_hbm, dst_v, ones_v, zeros_v, deg_sh):
        cid = lax.axis_index("c")
        sid = lax.axis_index("s")
        wid = cid * NS + sid
        pltpu.sync_copy(dst_hbm.at[wid], dst_v)
        one = jnp.ones((L,), jnp.float32)
        zero = jnp.zeros((L,), jnp.float32)
        for r in range(CHUNK // L):
            ones_v[pl.ds(r * L, L)] = one
            zeros_v[pl.ds(r * L, L)] = zero

        base = sid * zrows
        for k in range(zrows // CHUNK):
            pltpu.sync_copy(zeros_v, deg_sh.at[pl.ds(base + k * CHUNK, CHUNK)])
        plsc.subcore_barrier()

        @pl.loop(0, n_chunks)
        def _scat(j):
            pltpu.sync_copy(ones_v, deg_sh.at[dst_v.at[j]], add=True)

        plsc.subcore_barrier()
        for k in range(zrows // CHUNK):
            sl = pl.ds(base + k * CHUNK, CHUNK)
            pltpu.sync_copy(deg_sh.at[sl], out_hbm.at[cid, sl])

    return degk


def _make_propagate(n_nodes, rows_pad, n_chunks, d):
    zrows = rows_pad // NS
    mesh = plsc.VectorSubcoreMesh(core_axis_name="c", subcore_axis_name="s")

    @functools.partial(
        pl.kernel,
        out_type=jax.ShapeDtypeStruct((NC, rows_pad, d), jnp.float32),
        mesh=mesh,
        scratch_types=[
            pltpu.VMEM((n_chunks, CHUNK), jnp.int32),
            pltpu.VMEM((n_chunks, CHUNK), jnp.int32),
            pltpu.VMEM((CHUNK, d), jnp.float32),
            pltpu.VMEM_SHARED((rows_pad, d), jnp.float32),
            pltpu.SemaphoreType.DMA,
        ],
    )
    def prop(h_hbm, src_hbm, dst_hbm, out_hbm, src_v, dst_v, rows_v, acc_sh, sem):
        cid = lax.axis_index("c")
        sid = lax.axis_index("s")
        wid = cid * NS + sid
        pltpu.sync_copy(src_hbm.at[wid], src_v)
        pltpu.sync_copy(dst_hbm.at[wid], dst_v)

        zero = jnp.zeros((L,), jnp.float32)

        @pl.loop(0, CHUNK)
        def _zero(r):
            for c in range(d // L):
                rows_v[r, pl.ds(c * L, L)] = zero

        base = sid * zrows
        for k in range(zrows // CHUNK):
            pltpu.sync_copy(rows_v, acc_sh.at[pl.ds(base + k * CHUNK, CHUNK)])
        plsc.subcore_barrier()

        @pl.loop(0, n_chunks)
        def _edges(j):
            pltpu.async_copy(h_hbm.at[src_v.at[j]], rows_v, sem).wait()
            pltpu.sync_copy(rows_v, acc_sh.at[dst_v.at[j]], add=True)

        plsc.subcore_barrier()
        for k in range(zrows // CHUNK):
            sl = pl.ds(base + k * CHUNK, CHUNK)
            pltpu.sync_copy(acc_sh.at[sl], out_hbm.at[cid, sl])

    return prop


def _tc_call(body, out_shape, *args):
    return pl.pallas_call(body, out_shape=out_shape)(*args)


def kernel(x, edge_index, W1, b1, W2, b2, W3, b3, W4, b4):
    n, d_in = x.shape
    e = edge_index.shape[1]

    rows_pad = -(-(n + 1) // (NS * CHUNK)) * (NS * CHUNK)
    ept = -(-e // (NW * CHUNK)) * CHUNK
    tot = NW * ept

    src = edge_index[0].astype(jnp.int32)
    dst = edge_index[1].astype(jnp.int32)
    pad = tot - e
    ar = jnp.arange(pad, dtype=jnp.int32)
    src_t = jnp.concatenate([src, ar % n]).reshape(NW, ept // CHUNK, CHUNK)
    dst_t = jnp.concatenate([dst, n + ar % (rows_pad - n)]).reshape(
        NW, ept // CHUNK, CHUNK)
    n_chunks = ept // CHUNK

    degp = _make_degree(rows_pad, n_chunks)(dst_t)
    degp = degp.reshape(NC, rows_pad, 1)

    b1r = b1.reshape(1, -1)
    b2r = b2.reshape(1, -1)
    b3r = b3.reshape(1, -1)
    b4r = b4.reshape(1, -1)

    def dinv_of(deg_ref):
        deg = deg_ref[0, :n, :] + deg_ref[1, :n, :] + 1.0
        return lax.rsqrt(deg)

    def first_body(x_ref, w_ref, deg_ref, hs_ref):
        dinv = dinv_of(deg_ref)
        h = jnp.dot(x_ref[...], w_ref[...], precision=_HI,
                    preferred_element_type=jnp.float32)
        hs_ref[...] = h * dinv

    PD = 128

    def mid_body(dw, acc_ref, hsp_ref, deg_ref, b_ref, w_ref, hs_ref):
        dinv = dinv_of(deg_ref)
        t = (acc_ref[0, :n, :dw] + acc_ref[1, :n, :dw] + hsp_ref[:, :dw])
        xn = jnp.maximum(t * dinv + b_ref[...], 0.0)
        h = jnp.dot(xn, w_ref[...], precision=_HI,
                    preferred_element_type=jnp.float32)
        d_out = h.shape[1]
        hs = h * dinv
        if d_out < PD:
            hs = jnp.concatenate(
                [hs, jnp.zeros((n, PD - d_out), jnp.float32)], axis=1)
        hs_ref[...] = hs

    def last_body(dw, acc_ref, hsp_ref, deg_ref, b_ref, out_ref):
        dinv = dinv_of(deg_ref)
        t = (acc_ref[0, :n, :dw] + acc_ref[1, :n, :dw] + hsp_ref[:, :dw])
        out_ref[...] = t * dinv + b_ref[...]

    f32 = jnp.float32
    prop = _make_propagate(n, rows_pad, n_chunks, PD)
    d1, d2, d3, d4 = W1.shape[1], W2.shape[1], W3.shape[1], W4.shape[1]

    hs1 = _tc_call(first_body, jax.ShapeDtypeStruct((n, d1), f32),
                   x, W1, degp)
    acc1 = prop(hs1, src_t, dst_t)
    hs2 = _tc_call(functools.partial(mid_body, d1),
                   jax.ShapeDtypeStruct((n, PD), f32),
                   acc1, hs1, degp, b1r, W2)
    acc2 = prop(hs2, src_t, dst_t)
    hs3 = _tc_call(functools.partial(mid_body, d2),
                   jax.ShapeDtypeStruct((n, PD), f32),
                   acc2, hs2, degp, b2r, W3)
    acc3 = prop(hs3, src_t, dst_t)
    hs4 = _tc_call(functools.partial(mid_body, d3),
                   jax.ShapeDtypeStruct((n, PD), f32),
                   acc3, hs3, degp, b3r, W4)
    acc4 = prop(hs4, src_t, dst_t)
    out = _tc_call(functools.partial(last_body, d4),
                   jax.ShapeDtypeStruct((n, d4), f32),
                   acc4, hs4, degp, b4r)
    return out


# speedup vs baseline: 23.7505x; 1.3659x over previous
import functools

import jax
import jax.numpy as jnp
from jax import lax
from jax.experimental import pallas as pl
from jax.experimental.pallas import tpu as pltpu
from jax.experimental.pallas import tpu_sc as plsc

NC = 2
NS = 16
NW = NC * NS
L = 16
CHUNK = 128
DEGW = 16

_HI = jax.lax.Precision.HIGHEST


def _make_degree(rows_pad, n_chunks):
    zrows = rows_pad // NS
    mesh = plsc.VectorSubcoreMesh(core_axis_name="c", subcore_axis_name="s")

    @functools.partial(
        pl.kernel,
        out_type=jax.ShapeDtypeStruct((NC, rows_pad), jnp.float32),
        mesh=mesh,
        scratch_types=[
            pltpu.VMEM((n_chunks, CHUNK), jnp.int32),
            pltpu.VMEM((CHUNK,), jnp.float32),
            pltpu.VMEM((CHUNK,), jnp.float32),
            pltpu.VMEM_SHARED((rows_pad,), jnp.float32),
        ],
    )
    def degk(dst_hbm, out_hbm, dst_v, ones_v, zeros_v, deg_sh):
        cid = lax.axis_index("c")
        sid = lax.axis_index("s")
        wid = cid * NS + sid
        pltpu.sync_copy(dst_hbm.at[wid], dst_v)
        one = jnp.ones((L,), jnp.float32)
        zero = jnp.zeros((L,), jnp.float32)
        for r in range(CHUNK // L):
            ones_v[pl.ds(r * L, L)] = one
            zeros_v[pl.ds(r * L, L)] = zero

        base = sid * zrows
        for k in range(zrows // CHUNK):
            pltpu.sync_copy(zeros_v, deg_sh.at[pl.ds(base + k * CHUNK, CHUNK)])
        plsc.subcore_barrier()

        @pl.loop(0, n_chunks)
        def _scat(j):
            pltpu.sync_copy(ones_v, deg_sh.at[dst_v.at[j]], add=True)

        plsc.subcore_barrier()
        for k in range(zrows // CHUNK):
            sl = pl.ds(base + k * CHUNK, CHUNK)
            pltpu.sync_copy(deg_sh.at[sl], out_hbm.at[cid, sl])

    return degk


NBUF = 2
ZR = 64


def _make_propagate(n_nodes, rows_pad, n_chunks, d):
    assert n_chunks % NBUF == 0
    zrows = rows_pad // NS
    mesh = plsc.VectorSubcoreMesh(core_axis_name="c", subcore_axis_name="s")

    @functools.partial(
        pl.kernel,
        out_type=jax.ShapeDtypeStruct((NC, rows_pad, d), jnp.float32),
        mesh=mesh,
        scratch_types=[
            pltpu.VMEM((NBUF, CHUNK), jnp.int32),
            pltpu.VMEM((NBUF, CHUNK), jnp.int32),
            pltpu.VMEM((NBUF, CHUNK, d), jnp.float32),
            pltpu.VMEM((ZR, d), jnp.float32),
            pltpu.VMEM_SHARED((rows_pad, d), jnp.float32),
            pltpu.SemaphoreType.DMA((NBUF,)),
            pltpu.SemaphoreType.DMA((NBUF,)),
            pltpu.SemaphoreType.DMA((NBUF,)),
        ],
    )
    def prop(h_hbm, src_hbm, dst_hbm, out_hbm, src_v, dst_v, rows_v, zero_v,
             acc_sh, gsem, ssem, dsem):
        cid = lax.axis_index("c")
        sid = lax.axis_index("s")
        wid = cid * NS + sid

        for b in range(NBUF):
            pltpu.async_copy(src_hbm.at[wid, b], src_v.at[b], ssem.at[b])
            pltpu.async_copy(dst_hbm.at[wid, b], dst_v.at[b], dsem.at[b])
        pltpu.make_async_copy(src_hbm.at[wid, 0], src_v.at[0],
                              ssem.at[0]).wait()
        pltpu.async_copy(h_hbm.at[src_v.at[0]], rows_v.at[0], gsem.at[0])

        zero = jnp.zeros((L,), jnp.float32)

        @pl.loop(0, ZR)
        def _zfill(r):
            for c in range(d // L):
                zero_v[r, pl.ds(c * L, L)] = zero

        base = sid * zrows
        for k in range(zrows // ZR):
            pltpu.sync_copy(zero_v, acc_sh.at[pl.ds(base + k * ZR, ZR)])
        plsc.subcore_barrier()

        @pl.loop(0, n_chunks, step=NBUF)
        def _edges(j0):
            for b in range(NBUF):
                j = j0 + b
                o = 1 - b

                @pl.when(j + 1 < n_chunks)
                def _g_next():
                    pltpu.make_async_copy(src_hbm.at[wid, 0], src_v.at[o],
                                          ssem.at[o]).wait()
                    pltpu.async_copy(h_hbm.at[src_v.at[o]], rows_v.at[o],
                                     gsem.at[o])

                pltpu.make_async_copy(
                    h_hbm.at[src_v.at[b]], rows_v.at[b], gsem.at[b]).wait()
                pltpu.make_async_copy(dst_hbm.at[wid, 0], dst_v.at[b],
                                      dsem.at[b]).wait()
                pltpu.sync_copy(rows_v.at[b], acc_sh.at[dst_v.at[b]],
                                add=True)

                @pl.when(j + 2 < n_chunks)
                def _i_next():
                    pltpu.async_copy(src_hbm.at[wid, j + 2], src_v.at[b],
                                     ssem.at[b])
                    pltpu.async_copy(dst_hbm.at[wid, j + 2], dst_v.at[b],
                                     dsem.at[b])

        plsc.subcore_barrier()
        sl = pl.ds(base, zrows)
        pltpu.sync_copy(acc_sh.at[sl], out_hbm.at[cid, sl])

    return prop


def _tc_call(body, out_shape, *args):
    return pl.pallas_call(body, out_shape=out_shape)(*args)


def kernel(x, edge_index, W1, b1, W2, b2, W3, b3, W4, b4):
    n, d_in = x.shape
    e = edge_index.shape[1]

    rows_pad = -(-(n + 1) // (NS * CHUNK)) * (NS * CHUNK)
    ept = -(--(-e // (NW * CHUNK)) // NBUF) * NBUF * CHUNK
    tot = NW * ept

    src = edge_index[0].astype(jnp.int32)
    dst = edge_index[1].astype(jnp.int32)
    pad = tot - e
    ar = jnp.arange(pad, dtype=jnp.int32)
    src_t = jnp.concatenate([src, ar % n]).reshape(NW, ept // CHUNK, CHUNK)
    dst_t = jnp.concatenate([dst, n + ar % (rows_pad - n)]).reshape(
        NW, ept // CHUNK, CHUNK)
    n_chunks = ept // CHUNK

    degp = _make_degree(rows_pad, n_chunks)(dst_t)
    degp = degp.reshape(NC, rows_pad, 1)

    b1r = b1.reshape(1, -1)
    b2r = b2.reshape(1, -1)
    b3r = b3.reshape(1, -1)
    b4r = b4.reshape(1, -1)

    def dinv_of(deg_ref):
        deg = deg_ref[0, :n, :] + deg_ref[1, :n, :] + 1.0
        return lax.rsqrt(deg)

    def first_body(x_ref, w_ref, deg_ref, hs_ref):
        dinv = dinv_of(deg_ref)
        h = jnp.dot(x_ref[...], w_ref[...], precision=_HI,
                    preferred_element_type=jnp.float32)
        hs_ref[...] = h * dinv

    PD = 128

    def mid_body(dw, acc_ref, hsp_ref, deg_ref, b_ref, w_ref, hs_ref):
        dinv = dinv_of(deg_ref)
        t = (acc_ref[0, :n, :dw] + acc_ref[1, :n, :dw] + hsp_ref[:, :dw])
        xn = jnp.maximum(t * dinv + b_ref[...], 0.0)
        h = jnp.dot(xn, w_ref[...], precision=_HI,
                    preferred_element_type=jnp.float32)
        d_out = h.shape[1]
        hs = h * dinv
        if d_out < PD:
            hs = jnp.concatenate(
                [hs, jnp.zeros((n, PD - d_out), jnp.float32)], axis=1)
        hs_ref[...] = hs

    def last_body(dw, acc_ref, hsp_ref, deg_ref, b_ref, out_ref):
        dinv = dinv_of(deg_ref)
        t = (acc_ref[0, :n, :dw] + acc_ref[1, :n, :dw] + hsp_ref[:, :dw])
        out_ref[...] = t * dinv + b_ref[...]

    f32 = jnp.float32
    prop = _make_propagate(n, rows_pad, n_chunks, PD)
    d1, d2, d3, d4 = W1.shape[1], W2.shape[1], W3.shape[1], W4.shape[1]

    hs1 = _tc_call(first_body, jax.ShapeDtypeStruct((n, d1), f32),
                   x, W1, degp)
    acc1 = prop(hs1, src_t, dst_t)
    hs2 = _tc_call(functools.partial(mid_body, d1),
                   jax.ShapeDtypeStruct((n, PD), f32),
                   acc1, hs1, degp, b1r, W2)
    acc2 = prop(hs2, src_t, dst_t)
    hs3 = _tc_call(functools.partial(mid_body, d2),
                   jax.ShapeDtypeStruct((n, PD), f32),
                   acc2, hs2, degp, b2r, W3)
    acc3 = prop(hs3, src_t, dst_t)
    hs4 = _tc_call(functools.partial(mid_body, d3),
                   jax.ShapeDtypeStruct((n, PD), f32),
                   acc3, hs3, degp, b3r, W4)
    acc4 = prop(hs4, src_t, dst_t)
    out = _tc_call(functools.partial(last_body, d4),
                   jax.ShapeDtypeStruct((n, d4), f32),
                   acc4, hs4, degp, b4r)
    return out


# speedup vs baseline: 24.2571x; 1.0213x over previous
import functools

import jax
import jax.numpy as jnp
from jax import lax
from jax.experimental import pallas as pl
from jax.experimental.pallas import tpu as pltpu
from jax.experimental.pallas import tpu_sc as plsc

NC = 2
NS = 16
NW = NC * NS
L = 16
CHUNK = 128
DEGW = 16

_HI = jax.lax.Precision.HIGHEST


def _chunks_of(total, size):
    out, off = [], 0
    while off < total:
        out.append((off, min(size, total - off)))
        off += size
    return out


def _make_degree(rows_pad, n_chunks):
    zrows = rows_pad // NS
    mesh = plsc.VectorSubcoreMesh(core_axis_name="c", subcore_axis_name="s")

    @functools.partial(
        pl.kernel,
        out_type=jax.ShapeDtypeStruct((NC * rows_pad,), jnp.float32),
        mesh=mesh,
        scratch_types=[
            pltpu.VMEM((n_chunks, CHUNK), jnp.int32),
            pltpu.VMEM((CHUNK,), jnp.float32),
            pltpu.VMEM((CHUNK,), jnp.float32),
            pltpu.VMEM_SHARED((rows_pad,), jnp.float32),
        ],
    )
    def degk(dst_hbm, out_hbm, dst_v, ones_v, zeros_v, deg_sh):
        cid = lax.axis_index("c")
        sid = lax.axis_index("s")
        wid = cid * NS + sid
        pltpu.sync_copy(dst_hbm.at[wid], dst_v)
        one = jnp.ones((L,), jnp.float32)
        zero = jnp.zeros((L,), jnp.float32)
        for r in range(CHUNK // L):
            ones_v[pl.ds(r * L, L)] = one
            zeros_v[pl.ds(r * L, L)] = zero

        base = sid * zrows
        for off, ln in _chunks_of(zrows, CHUNK):
            pltpu.sync_copy(zeros_v.at[pl.ds(0, ln)],
                            deg_sh.at[pl.ds(base + off, ln)])
        plsc.subcore_barrier()

        @pl.loop(0, n_chunks)
        def _scat(j):
            pltpu.sync_copy(ones_v, deg_sh.at[dst_v.at[j]], add=True)

        plsc.subcore_barrier()
        for off, ln in _chunks_of(zrows, CHUNK):
            pltpu.sync_copy(
                deg_sh.at[pl.ds(base + off, ln)],
                out_hbm.at[pl.ds(cid * rows_pad + base + off, ln)])

    return degk


NBUF = 3


def _make_propagate(n_nodes, rows_pad, n_chunks, d):
    assert n_chunks % NBUF == 0
    zrows = rows_pad // NS
    mesh = plsc.VectorSubcoreMesh(core_axis_name="c", subcore_axis_name="s")

    @functools.partial(
        pl.kernel,
        out_type=jax.ShapeDtypeStruct((NC, rows_pad, d), jnp.float32),
        mesh=mesh,
        scratch_types=[
            pltpu.VMEM((NBUF, CHUNK), jnp.int32),
            pltpu.VMEM((NBUF, CHUNK), jnp.int32),
            pltpu.VMEM((NBUF, CHUNK, d), jnp.float32),
            pltpu.VMEM_SHARED((rows_pad, d), jnp.float32),
            pltpu.SemaphoreType.DMA((NBUF,)),
            pltpu.SemaphoreType.DMA((NBUF,)),
            pltpu.SemaphoreType.DMA((NBUF,)),
        ],
    )
    def prop(h_hbm, src_hbm, dst_hbm, out_hbm, src_v, dst_v, rows_v,
             acc_sh, gsem, ssem, dsem):
        cid = lax.axis_index("c")
        sid = lax.axis_index("s")
        wid = cid * NS + sid

        for b in range(NBUF):
            pltpu.async_copy(src_hbm.at[wid, b], src_v.at[b], ssem.at[b])
            pltpu.async_copy(dst_hbm.at[wid, b], dst_v.at[b], dsem.at[b])
        for b in range(2):
            pltpu.make_async_copy(src_hbm.at[wid, 0], src_v.at[b],
                                  ssem.at[b]).wait()
            pltpu.async_copy(h_hbm.at[src_v.at[b]], rows_v.at[b], gsem.at[b])

        zero = jnp.zeros((L,), jnp.float32)

        @pl.loop(0, CHUNK)
        def _zfill(r):
            for c in range(d // L):
                rows_v[2, r, pl.ds(c * L, L)] = zero

        base = sid * zrows
        for off, ln in _chunks_of(zrows, CHUNK):
            pltpu.sync_copy(rows_v.at[2, pl.ds(0, ln)],
                            acc_sh.at[pl.ds(base + off, ln)])
        plsc.subcore_barrier()

        @pl.loop(0, n_chunks, step=NBUF)
        def _edges(j0):
            for b in range(NBUF):
                j = j0 + b
                nx = (b + 2) % NBUF

                @pl.when(j + 2 < n_chunks)
                def _g_next():
                    pltpu.make_async_copy(src_hbm.at[wid, 0], src_v.at[nx],
                                          ssem.at[nx]).wait()
                    pltpu.async_copy(h_hbm.at[src_v.at[nx]], rows_v.at[nx],
                                     gsem.at[nx])

                pltpu.make_async_copy(
                    h_hbm.at[src_v.at[b]], rows_v.at[b], gsem.at[b]).wait()
                pltpu.make_async_copy(dst_hbm.at[wid, 0], dst_v.at[b],
                                      dsem.at[b]).wait()
                pltpu.sync_copy(rows_v.at[b], acc_sh.at[dst_v.at[b]],
                                add=True)

                @pl.when(j + NBUF < n_chunks)
                def _i_next():
                    pltpu.async_copy(src_hbm.at[wid, j + NBUF], src_v.at[b],
                                     ssem.at[b])
                    pltpu.async_copy(dst_hbm.at[wid, j + NBUF], dst_v.at[b],
                                     dsem.at[b])

        plsc.subcore_barrier()
        sl = pl.ds(base, zrows)
        pltpu.sync_copy(acc_sh.at[sl], out_hbm.at[cid, sl])

    return prop


def _tc_call(body, out_shape, *args):
    return pl.pallas_call(body, out_shape=out_shape)(*args)


def kernel(x, edge_index, W1, b1, W2, b2, W3, b3, W4, b4):
    n, d_in = x.shape
    e = edge_index.shape[1]

    rows_pad = -(-(n + 1) // (NS * 8)) * (NS * 8)
    ept = -(--(-e // (NW * CHUNK)) // NBUF) * NBUF * CHUNK
    tot = NW * ept

    src = edge_index[0].astype(jnp.int32)
    dst = edge_index[1].astype(jnp.int32)
    pad = tot - e
    ar = jnp.arange(pad, dtype=jnp.int32)
    src_t = jnp.concatenate([src, ar % n]).reshape(NW, ept // CHUNK, CHUNK)
    dst_t = jnp.concatenate([dst, n + ar % (rows_pad - n)]).reshape(
        NW, ept // CHUNK, CHUNK)
    n_chunks = ept // CHUNK

    rows_pad_deg = -(-(n + 1) // (NS * CHUNK)) * (NS * CHUNK)
    degp = _make_degree(rows_pad_deg, n_chunks)(dst_t)
    degp = degp.reshape(NC, rows_pad_deg, 1)

    b1r = b1.reshape(1, -1)
    b2r = b2.reshape(1, -1)
    b3r = b3.reshape(1, -1)
    b4r = b4.reshape(1, -1)

    def dinv_of(deg_ref):
        deg = deg_ref[0, :n, :] + deg_ref[1, :n, :] + 1.0
        return lax.rsqrt(deg)

    def first_body(x_ref, w_ref, deg_ref, hs_ref):
        dinv = dinv_of(deg_ref)
        h = jnp.dot(x_ref[...], w_ref[...], precision=_HI,
                    preferred_element_type=jnp.float32)
        hs_ref[...] = h * dinv

    PD = 128

    def mid_body(dw, acc_ref, hsp_ref, deg_ref, b_ref, w_ref, hs_ref):
        dinv = dinv_of(deg_ref)
        t = (acc_ref[0, :n, :dw] + acc_ref[1, :n, :dw] + hsp_ref[:, :dw])
        xn = jnp.maximum(t * dinv + b_ref[...], 0.0)
        h = jnp.dot(xn, w_ref[...], precision=_HI,
                    preferred_element_type=jnp.float32)
        d_out = h.shape[1]
        hs = h * dinv
        if d_out < PD:
            hs = jnp.concatenate(
                [hs, jnp.zeros((n, PD - d_out), jnp.float32)], axis=1)
        hs_ref[...] = hs

    def last_body(dw, acc_ref, hsp_ref, deg_ref, b_ref, out_ref):
        dinv = dinv_of(deg_ref)
        t = (acc_ref[0, :n, :dw] + acc_ref[1, :n, :dw] + hsp_ref[:, :dw])
        out_ref[...] = t * dinv + b_ref[...]

    f32 = jnp.float32
    prop = _make_propagate(n, rows_pad, n_chunks, PD)
    d1, d2, d3, d4 = W1.shape[1], W2.shape[1], W3.shape[1], W4.shape[1]

    hs1 = _tc_call(first_body, jax.ShapeDtypeStruct((n, d1), f32),
                   x, W1, degp)
    acc1 = prop(hs1, src_t, dst_t)
    hs2 = _tc_call(functools.partial(mid_body, d1),
                   jax.ShapeDtypeStruct((n, PD), f32),
                   acc1, hs1, degp, b1r, W2)
    acc2 = prop(hs2, src_t, dst_t)
    hs3 = _tc_call(functools.partial(mid_body, d2),
                   jax.ShapeDtypeStruct((n, PD), f32),
                   acc2, hs2, degp, b2r, W3)
    acc3 = prop(hs3, src_t, dst_t)
    hs4 = _tc_call(functools.partial(mid_body, d3),
                   jax.ShapeDtypeStruct((n, PD), f32),
                   acc3, hs3, degp, b3r, W4)
    acc4 = prop(hs4, src_t, dst_t)
    out = _tc_call(functools.partial(last_body, d4),
                   jax.ShapeDtypeStruct((n, d4), f32),
                   acc4, hs4, degp, b4r)
    return out


# speedup vs baseline: 28.5077x; 1.1752x over previous
import functools

import jax
import jax.numpy as jnp
from jax import lax
from jax.experimental import pallas as pl
from jax.experimental.pallas import tpu as pltpu
from jax.experimental.pallas import tpu_sc as plsc

NC = 2
NS = 16
NW = NC * NS
L = 16
CHUNK = 128
DEGW = 16

_HI = jax.lax.Precision.HIGHEST


def _chunks_of(total, size):
    out, off = [], 0
    while off < total:
        out.append((off, min(size, total - off)))
        off += size
    return out


def _make_degree(rows_pad, n_chunks):
    zrows = rows_pad // NS
    mesh = plsc.VectorSubcoreMesh(core_axis_name="c", subcore_axis_name="s")

    @functools.partial(
        pl.kernel,
        out_type=jax.ShapeDtypeStruct((NC * rows_pad,), jnp.float32),
        mesh=mesh,
        scratch_types=[
            pltpu.VMEM((n_chunks, CHUNK), jnp.int32),
            pltpu.VMEM((CHUNK,), jnp.float32),
            pltpu.VMEM((CHUNK,), jnp.float32),
            pltpu.VMEM_SHARED((rows_pad,), jnp.float32),
        ],
    )
    def degk(dst_hbm, out_hbm, dst_v, ones_v, zeros_v, deg_sh):
        cid = lax.axis_index("c")
        sid = lax.axis_index("s")
        wid = cid * NS + sid
        pltpu.sync_copy(dst_hbm.at[wid], dst_v)
        one = jnp.ones((L,), jnp.float32)
        zero = jnp.zeros((L,), jnp.float32)
        for r in range(CHUNK // L):
            ones_v[pl.ds(r * L, L)] = one
            zeros_v[pl.ds(r * L, L)] = zero

        base = sid * zrows
        for off, ln in _chunks_of(zrows, CHUNK):
            pltpu.sync_copy(zeros_v.at[pl.ds(0, ln)],
                            deg_sh.at[pl.ds(base + off, ln)])
        plsc.subcore_barrier()

        @pl.loop(0, n_chunks)
        def _scat(j):
            pltpu.sync_copy(ones_v, deg_sh.at[dst_v.at[j]], add=True)

        plsc.subcore_barrier()
        for off, ln in _chunks_of(zrows, CHUNK):
            pltpu.sync_copy(
                deg_sh.at[pl.ds(base + off, ln)],
                out_hbm.at[pl.ds(cid * rows_pad + base + off, ln)])

    return degk


NBUF = 3


def _make_propagate(n_nodes, rows_pad, n_chunks, d):
    assert n_chunks % NBUF == 0
    zrows = rows_pad // NS
    mesh = plsc.VectorSubcoreMesh(core_axis_name="c", subcore_axis_name="s")

    @functools.partial(
        pl.kernel,
        out_type=jax.ShapeDtypeStruct((NC, rows_pad, d), jnp.float32),
        mesh=mesh,
        scratch_types=[
            pltpu.VMEM((NBUF, CHUNK), jnp.int32),
            pltpu.VMEM((NBUF, CHUNK), jnp.int32),
            pltpu.VMEM((NBUF, CHUNK, d), jnp.float32),
            pltpu.VMEM_SHARED((rows_pad, d), jnp.float32),
            pltpu.SemaphoreType.DMA((NBUF,)),
            pltpu.SemaphoreType.DMA((NBUF,)),
            pltpu.SemaphoreType.DMA((NBUF,)),
            pltpu.SemaphoreType.DMA((NBUF,)),
        ],
    )
    def prop(h_hbm, src_hbm, dst_hbm, out_hbm, src_v, dst_v, rows_v,
             acc_sh, gsem, ssem, dsem, csem):
        cid = lax.axis_index("c")
        sid = lax.axis_index("s")
        wid = cid * NS + sid

        def scat_desc(b):
            return pltpu.make_async_copy(rows_v.at[b],
                                         acc_sh.at[dst_v.at[b]], csem.at[b])

        for b in range(NBUF):
            pltpu.async_copy(src_hbm.at[wid, b], src_v.at[b], ssem.at[b])
            pltpu.async_copy(dst_hbm.at[wid, b], dst_v.at[b], dsem.at[b])
        for b in range(2):
            pltpu.make_async_copy(src_hbm.at[wid, 0], src_v.at[b],
                                  ssem.at[b]).wait()
            pltpu.async_copy(h_hbm.at[src_v.at[b]], rows_v.at[b], gsem.at[b])

        zero = jnp.zeros((L,), jnp.float32)

        @pl.loop(0, CHUNK)
        def _zfill(r):
            for c in range(d // L):
                rows_v[2, r, pl.ds(c * L, L)] = zero

        base = sid * zrows
        for off, ln in _chunks_of(zrows, CHUNK):
            pltpu.sync_copy(rows_v.at[2, pl.ds(0, ln)],
                            acc_sh.at[pl.ds(base + off, ln)])
        plsc.subcore_barrier()

        @pl.loop(0, n_chunks, step=NBUF)
        def _edges(j0):
            for b in range(NBUF):
                j = j0 + b
                nx = (b + 2) % NBUF

                if b == 0:
                    @pl.when(j >= 1)
                    def _dr():
                        scat_desc(nx).wait()
                else:
                    scat_desc(nx).wait()

                @pl.when(jnp.logical_and(j >= 1, j + 2 < n_chunks))
                def _d_next():
                    pltpu.async_copy(dst_hbm.at[wid, j + 2], dst_v.at[nx],
                                     dsem.at[nx])

                @pl.when(j + 2 < n_chunks)
                def _g_next():
                    pltpu.make_async_copy(src_hbm.at[wid, 0], src_v.at[nx],
                                          ssem.at[nx]).wait()
                    pltpu.async_copy(h_hbm.at[src_v.at[nx]], rows_v.at[nx],
                                     gsem.at[nx])

                pltpu.make_async_copy(
                    h_hbm.at[src_v.at[b]], rows_v.at[b], gsem.at[b]).wait()

                @pl.when(j + NBUF < n_chunks)
                def _s_next():
                    pltpu.async_copy(src_hbm.at[wid, j + NBUF], src_v.at[b],
                                     ssem.at[b])

                pltpu.make_async_copy(dst_hbm.at[wid, 0], dst_v.at[b],
                                      dsem.at[b]).wait()
                pltpu.async_copy(rows_v.at[b], acc_sh.at[dst_v.at[b]],
                                 csem.at[b], add=True)

        scat_desc((n_chunks - 1) % NBUF).wait()
        plsc.subcore_barrier()
        sl = pl.ds(base, zrows)
        pltpu.sync_copy(acc_sh.at[sl], out_hbm.at[cid, sl])

    return prop


def _tc_call(body, out_shape, *args):
    return pl.pallas_call(body, out_shape=out_shape)(*args)


def kernel(x, edge_index, W1, b1, W2, b2, W3, b3, W4, b4):
    n, d_in = x.shape
    e = edge_index.shape[1]

    rows_pad = -(-(n + 1) // (NS * 8)) * (NS * 8)
    ept = -(--(-e // (NW * CHUNK)) // NBUF) * NBUF * CHUNK
    tot = NW * ept

    src = edge_index[0].astype(jnp.int32)
    dst = edge_index[1].astype(jnp.int32)
    pad = tot - e
    ar = jnp.arange(pad, dtype=jnp.int32)
    src_t = jnp.concatenate([src, ar % n]).reshape(NW, ept // CHUNK, CHUNK)
    dst_t = jnp.concatenate([dst, n + ar % (rows_pad - n)]).reshape(
        NW, ept // CHUNK, CHUNK)
    n_chunks = ept // CHUNK

    rows_pad_deg = -(-(n + 1) // (NS * CHUNK)) * (NS * CHUNK)
    degp = _make_degree(rows_pad_deg, n_chunks)(dst_t)
    degp = degp.reshape(NC, rows_pad_deg, 1)

    b1r = b1.reshape(1, -1)
    b2r = b2.reshape(1, -1)
    b3r = b3.reshape(1, -1)
    b4r = b4.reshape(1, -1)

    def dinv_of(deg_ref):
        deg = deg_ref[0, :n, :] + deg_ref[1, :n, :] + 1.0
        return lax.rsqrt(deg)

    def first_body(x_ref, w_ref, deg_ref, hs_ref):
        dinv = dinv_of(deg_ref)
        h = jnp.dot(x_ref[...], w_ref[...], precision=_HI,
                    preferred_element_type=jnp.float32)
        hs_ref[...] = h * dinv

    PD = 128

    def mid_body(dw, acc_ref, hsp_ref, deg_ref, b_ref, w_ref, hs_ref):
        dinv = dinv_of(deg_ref)
        t = (acc_ref[0, :n, :dw] + acc_ref[1, :n, :dw] + hsp_ref[:, :dw])
        xn = jnp.maximum(t * dinv + b_ref[...], 0.0)
        h = jnp.dot(xn, w_ref[...], precision=_HI,
                    preferred_element_type=jnp.float32)
        d_out = h.shape[1]
        hs = h * dinv
        if d_out < PD:
            hs = jnp.concatenate(
                [hs, jnp.zeros((n, PD - d_out), jnp.float32)], axis=1)
        hs_ref[...] = hs

    def last_body(dw, acc_ref, hsp_ref, deg_ref, b_ref, out_ref):
        dinv = dinv_of(deg_ref)
        t = (acc_ref[0, :n, :dw] + acc_ref[1, :n, :dw] + hsp_ref[:, :dw])
        out_ref[...] = t * dinv + b_ref[...]

    f32 = jnp.float32
    prop = _make_propagate(n, rows_pad, n_chunks, PD)
    d1, d2, d3, d4 = W1.shape[1], W2.shape[1], W3.shape[1], W4.shape[1]

    hs1 = _tc_call(first_body, jax.ShapeDtypeStruct((n, d1), f32),
                   x, W1, degp)
    acc1 = prop(hs1, src_t, dst_t)
    hs2 = _tc_call(functools.partial(mid_body, d1),
                   jax.ShapeDtypeStruct((n, PD), f32),
                   acc1, hs1, degp, b1r, W2)
    acc2 = prop(hs2, src_t, dst_t)
    hs3 = _tc_call(functools.partial(mid_body, d2),
                   jax.ShapeDtypeStruct((n, PD), f32),
                   acc2, hs2, degp, b2r, W3)
    acc3 = prop(hs3, src_t, dst_t)
    hs4 = _tc_call(functools.partial(mid_body, d3),
                   jax.ShapeDtypeStruct((n, PD), f32),
                   acc3, hs3, degp, b3r, W4)
    acc4 = prop(hs4, src_t, dst_t)
    out = _tc_call(functools.partial(last_body, d4),
                   jax.ShapeDtypeStruct((n, d4), f32),
                   acc4, hs4, degp, b4r)
    return out


# speedup vs baseline: 29.0500x; 1.0190x over previous
import functools

import jax
import jax.numpy as jnp
from jax import lax
from jax.experimental import pallas as pl
from jax.experimental.pallas import tpu as pltpu
from jax.experimental.pallas import tpu_sc as plsc

NC = 2
NS = 16
NW = NC * NS
L = 16
CHUNK = 128
DEGW = 16

_HI = jax.lax.Precision.HIGHEST


def _chunks_of(total, size):
    out, off = [], 0
    while off < total:
        out.append((off, min(size, total - off)))
        off += size
    return out


def _make_degree(rows_pad, n_chunks):
    zrows = rows_pad // NS
    mesh = plsc.VectorSubcoreMesh(core_axis_name="c", subcore_axis_name="s")

    @functools.partial(
        pl.kernel,
        out_type=jax.ShapeDtypeStruct((NC * rows_pad,), jnp.float32),
        mesh=mesh,
        scratch_types=[
            pltpu.VMEM((n_chunks, CHUNK), jnp.int32),
            pltpu.VMEM((CHUNK,), jnp.float32),
            pltpu.VMEM((CHUNK,), jnp.float32),
            pltpu.VMEM_SHARED((rows_pad,), jnp.float32),
        ],
    )
    def degk(dst_hbm, out_hbm, dst_v, ones_v, zeros_v, deg_sh):
        cid = lax.axis_index("c")
        sid = lax.axis_index("s")
        wid = cid * NS + sid
        pltpu.sync_copy(dst_hbm.at[wid], dst_v)
        one = jnp.ones((L,), jnp.float32)
        zero = jnp.zeros((L,), jnp.float32)
        for r in range(CHUNK // L):
            ones_v[pl.ds(r * L, L)] = one
            zeros_v[pl.ds(r * L, L)] = zero

        base = sid * zrows
        for off, ln in _chunks_of(zrows, CHUNK):
            pltpu.sync_copy(zeros_v.at[pl.ds(0, ln)],
                            deg_sh.at[pl.ds(base + off, ln)])
        plsc.subcore_barrier()

        @pl.loop(0, n_chunks)
        def _scat(j):
            pltpu.sync_copy(ones_v, deg_sh.at[dst_v.at[j]], add=True)

        plsc.subcore_barrier()
        for off, ln in _chunks_of(zrows, CHUNK):
            pltpu.sync_copy(
                deg_sh.at[pl.ds(base + off, ln)],
                out_hbm.at[pl.ds(cid * rows_pad + base + off, ln)])

    return degk


NBUF = 3


def _make_propagate(n_nodes, rows_pad, n_chunks, d):
    assert n_chunks >= 2 * NBUF
    zrows = rows_pad // NS
    mesh = plsc.VectorSubcoreMesh(core_axis_name="c", subcore_axis_name="s")

    @functools.partial(
        pl.kernel,
        out_type=jax.ShapeDtypeStruct((NC, rows_pad, d), jnp.float32),
        mesh=mesh,
        scratch_types=[
            pltpu.VMEM((NBUF, CHUNK), jnp.int32),
            pltpu.VMEM((NBUF, CHUNK), jnp.int32),
            pltpu.VMEM((NBUF, CHUNK, d), jnp.float32),
            pltpu.VMEM_SHARED((rows_pad, d), jnp.float32),
            pltpu.SemaphoreType.DMA((NBUF,)),
            pltpu.SemaphoreType.DMA((NBUF,)),
            pltpu.SemaphoreType.DMA((NBUF,)),
            pltpu.SemaphoreType.DMA((NBUF,)),
        ],
    )
    def prop(h_hbm, src_hbm, dst_hbm, out_hbm, src_v, dst_v, rows_v,
             acc_sh, gsem, ssem, dsem, csem):
        cid = lax.axis_index("c")
        sid = lax.axis_index("s")
        wid = cid * NS + sid

        def scat_desc(b):
            return pltpu.make_async_copy(rows_v.at[b],
                                         acc_sh.at[dst_v.at[b]], csem.at[b])

        for b in range(NBUF):
            pltpu.async_copy(src_hbm.at[wid, b], src_v.at[b], ssem.at[b])
            pltpu.async_copy(dst_hbm.at[wid, b], dst_v.at[b], dsem.at[b])
        for b in range(2):
            pltpu.make_async_copy(src_hbm.at[wid, 0], src_v.at[b],
                                  ssem.at[b]).wait()
            pltpu.async_copy(h_hbm.at[src_v.at[b]], rows_v.at[b], gsem.at[b])

        zero = jnp.zeros((L,), jnp.float32)

        @pl.loop(0, CHUNK)
        def _zfill(r):
            for c in range(d // L):
                rows_v[2, r, pl.ds(c * L, L)] = zero

        base = sid * zrows
        for off, ln in _chunks_of(zrows, CHUNK):
            pltpu.sync_copy(rows_v.at[2, pl.ds(0, ln)],
                            acc_sh.at[pl.ds(base + off, ln)])
        plsc.subcore_barrier()

        def when(cond, fn):
            if isinstance(cond, bool):
                if cond:
                    fn()
            else:
                pl.when(cond)(fn)

        def chunk_body(j, b, first):
            nx = (b + 2) % NBUF

            if first:
                when(j >= 1, lambda: scat_desc(nx).wait())
            else:
                scat_desc(nx).wait()

            def _d_next():
                pltpu.async_copy(dst_hbm.at[wid, j + 2], dst_v.at[nx],
                                 dsem.at[nx])

            if isinstance(j, int):
                when((j >= 1) and (j + 2 < n_chunks), _d_next)
            else:
                when(jnp.logical_and(j >= 1, j + 2 < n_chunks), _d_next)

            def _g_next():
                pltpu.make_async_copy(src_hbm.at[wid, 0], src_v.at[nx],
                                      ssem.at[nx]).wait()
                pltpu.async_copy(h_hbm.at[src_v.at[nx]], rows_v.at[nx],
                                 gsem.at[nx])

            when(j + 2 < n_chunks, _g_next)

            pltpu.make_async_copy(
                h_hbm.at[src_v.at[b]], rows_v.at[b], gsem.at[b]).wait()

            def _s_next():
                pltpu.async_copy(src_hbm.at[wid, j + NBUF], src_v.at[b],
                                 ssem.at[b])

            when(j + NBUF < n_chunks, _s_next)

            pltpu.make_async_copy(dst_hbm.at[wid, 0], dst_v.at[b],
                                  dsem.at[b]).wait()
            pltpu.async_copy(rows_v.at[b], acc_sh.at[dst_v.at[b]],
                             csem.at[b], add=True)

        n_main = (n_chunks // NBUF) * NBUF

        @pl.loop(0, n_main, step=NBUF)
        def _edges(j0):
            for b in range(NBUF):
                chunk_body(j0 + b, b, first=(b == 0))

        for j in range(n_main, n_chunks):
            chunk_body(j, j % NBUF, first=False)

        scat_desc((n_chunks - 1) % NBUF).wait()
        plsc.subcore_barrier()
        sl = pl.ds(base, zrows)
        pltpu.sync_copy(acc_sh.at[sl], out_hbm.at[cid, sl])

    return prop


def _tc_call(body, out_shape, *args):
    return pl.pallas_call(body, out_shape=out_shape)(*args)


def kernel(x, edge_index, W1, b1, W2, b2, W3, b3, W4, b4):
    n, d_in = x.shape
    e = edge_index.shape[1]

    rows_pad = -(-(n + 1) // (NS * 8)) * (NS * 8)
    ept = -(-e // (NW * CHUNK)) * CHUNK
    tot = NW * ept

    src = edge_index[0].astype(jnp.int32)
    dst = edge_index[1].astype(jnp.int32)
    pad = tot - e
    ar = jnp.arange(pad, dtype=jnp.int32)
    src_t = jnp.concatenate([src, ar % n]).reshape(NW, ept // CHUNK, CHUNK)
    dst_t = jnp.concatenate([dst, n + ar % (rows_pad - n)]).reshape(
        NW, ept // CHUNK, CHUNK)
    n_chunks = ept // CHUNK

    rows_pad_deg = -(-(n + 1) // (NS * CHUNK)) * (NS * CHUNK)
    degp = _make_degree(rows_pad_deg, n_chunks)(dst_t)
    degp = degp.reshape(NC, rows_pad_deg, 1)

    b1r = b1.reshape(1, -1)
    b2r = b2.reshape(1, -1)
    b3r = b3.reshape(1, -1)
    b4r = b4.reshape(1, -1)

    def dinv_of(deg_ref):
        deg = deg_ref[0, :n, :] + deg_ref[1, :n, :] + 1.0
        return lax.rsqrt(deg)

    def first_body(x_ref, w_ref, deg_ref, hs_ref):
        dinv = dinv_of(deg_ref)
        h = jnp.dot(x_ref[...], w_ref[...], precision=_HI,
                    preferred_element_type=jnp.float32)
        hs_ref[...] = h * dinv

    PD = 128

    def mid_body(dw, acc_ref, hsp_ref, deg_ref, b_ref, w_ref, hs_ref):
        dinv = dinv_of(deg_ref)
        t = (acc_ref[0, :n, :dw] + acc_ref[1, :n, :dw] + hsp_ref[:, :dw])
        xn = jnp.maximum(t * dinv + b_ref[...], 0.0)
        h = jnp.dot(xn, w_ref[...], precision=_HI,
                    preferred_element_type=jnp.float32)
        d_out = h.shape[1]
        hs = h * dinv
        if d_out < PD:
            hs = jnp.concatenate(
                [hs, jnp.zeros((n, PD - d_out), jnp.float32)], axis=1)
        hs_ref[...] = hs

    def last_body(dw, acc_ref, hsp_ref, deg_ref, b_ref, out_ref):
        dinv = dinv_of(deg_ref)
        t = (acc_ref[0, :n, :dw] + acc_ref[1, :n, :dw] + hsp_ref[:, :dw])
        out_ref[...] = t * dinv + b_ref[...]

    f32 = jnp.float32
    prop = _make_propagate(n, rows_pad, n_chunks, PD)
    d1, d2, d3, d4 = W1.shape[1], W2.shape[1], W3.shape[1], W4.shape[1]

    hs1 = _tc_call(first_body, jax.ShapeDtypeStruct((n, d1), f32),
                   x, W1, degp)
    acc1 = prop(hs1, src_t, dst_t)
    hs2 = _tc_call(functools.partial(mid_body, d1),
                   jax.ShapeDtypeStruct((n, PD), f32),
                   acc1, hs1, degp, b1r, W2)
    acc2 = prop(hs2, src_t, dst_t)
    hs3 = _tc_call(functools.partial(mid_body, d2),
                   jax.ShapeDtypeStruct((n, PD), f32),
                   acc2, hs2, degp, b2r, W3)
    acc3 = prop(hs3, src_t, dst_t)
    hs4 = _tc_call(functools.partial(mid_body, d3),
                   jax.ShapeDtypeStruct((n, PD), f32),
                   acc3, hs3, degp, b3r, W4)
    acc4 = prop(hs4, src_t, dst_t)
    out = _tc_call(functools.partial(last_body, d4),
                   jax.ShapeDtypeStruct((n, d4), f32),
                   acc4, hs4, degp, b4r)
    return out


# speedup vs baseline: 29.3753x; 1.0112x over previous
import functools

import jax
import jax.numpy as jnp
from jax import lax
from jax.experimental import pallas as pl
from jax.experimental.pallas import tpu as pltpu
from jax.experimental.pallas import tpu_sc as plsc

NC = 2
NS = 16
NW = NC * NS
L = 16
CHUNK = 128
DEGW = 16

_HI = jax.lax.Precision.HIGHEST


def _chunks_of(total, size):
    out, off = [], 0
    while off < total:
        out.append((off, min(size, total - off)))
        off += size
    return out


def _make_degree(rows_pad, n_chunks):
    zrows = rows_pad // NS
    mesh = plsc.VectorSubcoreMesh(core_axis_name="c", subcore_axis_name="s")

    @functools.partial(
        pl.kernel,
        out_type=jax.ShapeDtypeStruct((NC * rows_pad,), jnp.float32),
        mesh=mesh,
        scratch_types=[
            pltpu.VMEM((n_chunks, CHUNK), jnp.int32),
            pltpu.VMEM((CHUNK,), jnp.float32),
            pltpu.VMEM((CHUNK,), jnp.float32),
            pltpu.VMEM_SHARED((rows_pad,), jnp.float32),
            pltpu.SemaphoreType.DMA,
        ],
    )
    def degk(dst_hbm, out_hbm, dst_v, ones_v, zeros_v, deg_sh, dssem):
        cid = lax.axis_index("c")
        sid = lax.axis_index("s")
        wid = cid * NS + sid
        pltpu.sync_copy(dst_hbm.at[wid], dst_v)
        one = jnp.ones((L,), jnp.float32)
        zero = jnp.zeros((L,), jnp.float32)
        for r in range(CHUNK // L):
            ones_v[pl.ds(r * L, L)] = one
            zeros_v[pl.ds(r * L, L)] = zero

        base = sid * zrows
        for off, ln in _chunks_of(zrows, CHUNK):
            pltpu.sync_copy(zeros_v.at[pl.ds(0, ln)],
                            deg_sh.at[pl.ds(base + off, ln)])
        plsc.subcore_barrier()

        WAVE = 8
        n_main = (n_chunks // WAVE) * WAVE

        @pl.loop(0, n_main, step=WAVE)
        def _scat(j0):
            descs = [pltpu.async_copy(ones_v, deg_sh.at[dst_v.at[j0 + b]],
                                      dssem, add=True) for b in range(WAVE)]
            for de in descs:
                de.wait()

        tail = [pltpu.async_copy(ones_v, deg_sh.at[dst_v.at[j]], dssem,
                                 add=True) for j in range(n_main, n_chunks)]
        for de in tail:
            de.wait()

        plsc.subcore_barrier()
        for off, ln in _chunks_of(zrows, CHUNK):
            pltpu.sync_copy(
                deg_sh.at[pl.ds(base + off, ln)],
                out_hbm.at[pl.ds(cid * rows_pad + base + off, ln)])

    return degk


NBUF = 3


def _make_propagate(n_nodes, rows_pad, n_chunks, d):
    assert n_chunks >= 2 * NBUF
    zrows = rows_pad // NS
    mesh = plsc.VectorSubcoreMesh(core_axis_name="c", subcore_axis_name="s")

    @functools.partial(
        pl.kernel,
        out_type=jax.ShapeDtypeStruct((NC, rows_pad, d), jnp.float32),
        mesh=mesh,
        scratch_types=[
            pltpu.VMEM((NBUF, CHUNK), jnp.int32),
            pltpu.VMEM((NBUF, CHUNK), jnp.int32),
            pltpu.VMEM((NBUF, CHUNK, d), jnp.float32),
            pltpu.VMEM_SHARED((rows_pad, d), jnp.float32),
            pltpu.SemaphoreType.DMA((NBUF,)),
            pltpu.SemaphoreType.DMA((NBUF,)),
            pltpu.SemaphoreType.DMA((NBUF,)),
            pltpu.SemaphoreType.DMA((NBUF,)),
        ],
    )
    def prop(h_hbm, src_hbm, dst_hbm, out_hbm, src_v, dst_v, rows_v,
             acc_sh, gsem, ssem, dsem, csem):
        cid = lax.axis_index("c")
        sid = lax.axis_index("s")
        wid = cid * NS + sid

        def scat_desc(b):
            return pltpu.make_async_copy(rows_v.at[b],
                                         acc_sh.at[dst_v.at[b]], csem.at[b])

        for b in range(NBUF):
            pltpu.async_copy(src_hbm.at[wid, b], src_v.at[b], ssem.at[b])
            pltpu.async_copy(dst_hbm.at[wid, b], dst_v.at[b], dsem.at[b])
        for b in range(2):
            pltpu.make_async_copy(src_hbm.at[wid, 0], src_v.at[b],
                                  ssem.at[b]).wait()
            pltpu.async_copy(h_hbm.at[src_v.at[b]], rows_v.at[b], gsem.at[b])

        zero = jnp.zeros((L,), jnp.float32)

        @pl.loop(0, CHUNK)
        def _zfill(r):
            for c in range(d // L):
                rows_v[2, r, pl.ds(c * L, L)] = zero

        base = sid * zrows
        zdescs = []
        for off, ln in _chunks_of(zrows, CHUNK):
            zdescs.append(pltpu.async_copy(
                rows_v.at[2, pl.ds(0, ln)],
                acc_sh.at[pl.ds(base + off, ln)], csem.at[2]))
        for zd in zdescs:
            zd.wait()
        plsc.subcore_barrier()

        def when(cond, fn):
            if isinstance(cond, bool):
                if cond:
                    fn()
            else:
                pl.when(cond)(fn)

        def chunk_body(j, b, first):
            nx = (b + 2) % NBUF

            if first:
                when(j >= 1, lambda: scat_desc(nx).wait())
            else:
                scat_desc(nx).wait()

            def _d_next():
                pltpu.async_copy(dst_hbm.at[wid, j + 2], dst_v.at[nx],
                                 dsem.at[nx])

            if isinstance(j, int):
                when((j >= 1) and (j + 2 < n_chunks), _d_next)
            else:
                when(jnp.logical_and(j >= 1, j + 2 < n_chunks), _d_next)

            def _g_next():
                pltpu.make_async_copy(src_hbm.at[wid, 0], src_v.at[nx],
                                      ssem.at[nx]).wait()
                pltpu.async_copy(h_hbm.at[src_v.at[nx]], rows_v.at[nx],
                                 gsem.at[nx])

            when(j + 2 < n_chunks, _g_next)

            pltpu.make_async_copy(
                h_hbm.at[src_v.at[b]], rows_v.at[b], gsem.at[b]).wait()

            def _s_next():
                pltpu.async_copy(src_hbm.at[wid, j + NBUF], src_v.at[b],
                                 ssem.at[b])

            when(j + NBUF < n_chunks, _s_next)

            pltpu.make_async_copy(dst_hbm.at[wid, 0], dst_v.at[b],
                                  dsem.at[b]).wait()
            pltpu.async_copy(rows_v.at[b], acc_sh.at[dst_v.at[b]],
                             csem.at[b], add=True)

        n_main = (n_chunks // NBUF) * NBUF

        @pl.loop(0, n_main, step=NBUF)
        def _edges(j0):
            for b in range(NBUF):
                chunk_body(j0 + b, b, first=(b == 0))

        for j in range(n_main, n_chunks):
            chunk_body(j, j % NBUF, first=False)

        scat_desc((n_chunks - 1) % NBUF).wait()
        plsc.subcore_barrier()
        sl = pl.ds(base, zrows)
        pltpu.sync_copy(acc_sh.at[sl], out_hbm.at[cid, sl])

    return prop


def _tc_call(body, out_shape, *args):
    return pl.pallas_call(body, out_shape=out_shape)(*args)


def kernel(x, edge_index, W1, b1, W2, b2, W3, b3, W4, b4):
    n, d_in = x.shape
    e = edge_index.shape[1]

    rows_pad = -(-(n + 1) // (NS * 8)) * (NS * 8)
    ept = -(-e // (NW * CHUNK)) * CHUNK
    tot = NW * ept

    src = edge_index[0].astype(jnp.int32)
    dst = edge_index[1].astype(jnp.int32)
    pad = tot - e
    ar = jnp.arange(pad, dtype=jnp.int32)
    src_t = jnp.concatenate([src, ar % n]).reshape(NW, ept // CHUNK, CHUNK)
    dst_t = jnp.concatenate([dst, n + ar % (rows_pad - n)]).reshape(
        NW, ept // CHUNK, CHUNK)
    n_chunks = ept // CHUNK

    rows_pad_deg = -(-(n + 1) // (NS * CHUNK)) * (NS * CHUNK)
    degp = _make_degree(rows_pad_deg, n_chunks)(dst_t)
    degp = degp.reshape(NC, rows_pad_deg, 1)

    b1r = b1.reshape(1, -1)
    b2r = b2.reshape(1, -1)
    b3r = b3.reshape(1, -1)
    b4r = b4.reshape(1, -1)

    def dinv_of(deg_ref):
        deg = deg_ref[0, :n, :] + deg_ref[1, :n, :] + 1.0
        return lax.rsqrt(deg)

    def first_body(x_ref, w_ref, deg_ref, hs_ref):
        dinv = dinv_of(deg_ref)
        h = jnp.dot(x_ref[...], w_ref[...], precision=_HI,
                    preferred_element_type=jnp.float32)
        hs_ref[...] = h * dinv

    PD = 128

    def mid_body(dw, acc_ref, hsp_ref, deg_ref, b_ref, w_ref, hs_ref):
        dinv = dinv_of(deg_ref)
        t = (acc_ref[0, :n, :dw] + acc_ref[1, :n, :dw] + hsp_ref[:, :dw])
        xn = jnp.maximum(t * dinv + b_ref[...], 0.0)
        h = jnp.dot(xn, w_ref[...], precision=_HI,
                    preferred_element_type=jnp.float32)
        d_out = h.shape[1]
        hs = h * dinv
        if d_out < PD:
            hs = jnp.concatenate(
                [hs, jnp.zeros((n, PD - d_out), jnp.float32)], axis=1)
        hs_ref[...] = hs

    def last_body(dw, acc_ref, hsp_ref, deg_ref, b_ref, out_ref):
        dinv = dinv_of(deg_ref)
        t = (acc_ref[0, :n, :dw] + acc_ref[1, :n, :dw] + hsp_ref[:, :dw])
        out_ref[...] = t * dinv + b_ref[...]

    f32 = jnp.float32
    prop = _make_propagate(n, rows_pad, n_chunks, PD)
    d1, d2, d3, d4 = W1.shape[1], W2.shape[1], W3.shape[1], W4.shape[1]

    hs1 = _tc_call(first_body, jax.ShapeDtypeStruct((n, d1), f32),
                   x, W1, degp)
    acc1 = prop(hs1, src_t, dst_t)
    hs2 = _tc_call(functools.partial(mid_body, d1),
                   jax.ShapeDtypeStruct((n, PD), f32),
                   acc1, hs1, degp, b1r, W2)
    acc2 = prop(hs2, src_t, dst_t)
    hs3 = _tc_call(functools.partial(mid_body, d2),
                   jax.ShapeDtypeStruct((n, PD), f32),
                   acc2, hs2, degp, b2r, W3)
    acc3 = prop(hs3, src_t, dst_t)
    hs4 = _tc_call(functools.partial(mid_body, d3),
                   jax.ShapeDtypeStruct((n, PD), f32),
                   acc3, hs3, degp, b3r, W4)
    acc4 = prop(hs4, src_t, dst_t)
    out = _tc_call(functools.partial(last_body, d4),
                   jax.ShapeDtypeStruct((n, d4), f32),
                   acc4, hs4, degp, b4r)
    return out
